# baseline (speedup 1.0000x reference)
import jax
import jax.numpy as jnp
from jax.experimental import pallas as pl
from jax.experimental.pallas import tpu as pltpu

_ROWS = 128
_COLS = 100000
_SUP_STRIDE = 200
_SUP_LIMIT = 6400
_N_SUP = _SUP_LIMIT // _SUP_STRIDE
_WIN = 128


def _win_start(k):
    return (k * _SUP_STRIDE) // _WIN * _WIN


def _scatter_body(x_any, o_hbm, wins, sem_in, sem_out):
    del x_any
    for k in range(_N_SUP):
        pltpu.make_async_copy(
            o_hbm.at[:, pl.ds(_win_start(k), _WIN)], wins.at[k], sem_in.at[k]
        ).start()
    neg = jnp.full((_ROWS, 1), -jnp.inf, jnp.float32)
    for k in range(_N_SUP):
        pltpu.make_async_copy(
            o_hbm.at[:, pl.ds(_win_start(k), _WIN)], wins.at[k], sem_in.at[k]
        ).wait()
        off = k * _SUP_STRIDE - _win_start(k)
        wins[k, :, off : off + 1] = neg
        pltpu.make_async_copy(
            wins.at[k], o_hbm.at[:, pl.ds(_win_start(k), _WIN)], sem_out.at[k]
        ).start()
    for k in range(_N_SUP):
        pltpu.make_async_copy(
            wins.at[k], o_hbm.at[:, pl.ds(_win_start(k), _WIN)], sem_out.at[k]
        ).wait()


def kernel(scores):
    bulk = jnp.maximum(scores, -jnp.inf)
    return pl.pallas_call(
        _scatter_body,
        in_specs=[pl.BlockSpec(memory_space=pl.MemorySpace.ANY)],
        out_specs=pl.BlockSpec(memory_space=pl.MemorySpace.ANY),
        out_shape=jax.ShapeDtypeStruct((_ROWS, _COLS), scores.dtype),
        scratch_shapes=[
            pltpu.MemorySpace.VMEM((_N_SUP, _ROWS, _WIN), jnp.float32),
            pltpu.SemaphoreType.DMA((_N_SUP,)),
            pltpu.SemaphoreType.DMA((_N_SUP,)),
        ],
        input_output_aliases={0: 0},
    )(bulk)
